# baseline (speedup 1.0000x reference)
import jax
import jax.numpy as jnp
from jax.experimental import pallas as pl

_E = 8
_K = 2
_R = 8
_ALPHA = 32.0
_SCALING = _ALPHA / _R

_BT = 2048


def _moe_body(x_ref, m_ref, bg_ref, bf_ref, rep_ref, o_ref):
    xv = x_ref[...]
    h_all = jnp.dot(xv, m_ref[...], preferred_element_type=jnp.float32)
    h = h_all[:, : _E * _R]
    logits = h_all[:, _E * _R :] + bg_ref[...]

    col = jax.lax.broadcasted_iota(jnp.int32, logits.shape, 1).astype(jnp.float32)
    neg_inf = jnp.float32(-jnp.inf)
    big = jnp.float32(_E)

    m1 = jnp.max(logits, axis=-1, keepdims=True)
    i1 = jnp.min(jnp.where(logits == m1, col, big), axis=-1, keepdims=True)
    sel1 = col == i1
    l2 = jnp.where(sel1, neg_inf, logits)
    m2 = jnp.max(l2, axis=-1, keepdims=True)
    i2 = jnp.min(jnp.where(l2 == m2, col, big), axis=-1, keepdims=True)
    sel2 = col == i2

    p2 = jnp.exp(m2 - m1)
    w1 = 1.0 / (1.0 + p2)
    w2 = 1.0 - w1
    zero = jnp.float32(0.0)
    w_full = jnp.where(sel1, w1, zero) + jnp.where(sel2, w2, zero)

    w_rep = jnp.dot(w_full, rep_ref[...], preferred_element_type=jnp.float32)
    o_ref[...] = jnp.dot(h * w_rep, bf_ref[...], preferred_element_type=jnp.float32)


@jax.jit
def kernel(x, Wg, bg, A, B):
    T, D = x.shape
    E, R, _ = A.shape
    a_flat = A.reshape(E * R, D).T
    m = jnp.concatenate([a_flat, Wg.T], axis=1)
    b_flat = (B.transpose(0, 2, 1) * jnp.float32(_SCALING)).reshape(E * R, D)
    rep = jnp.repeat(jnp.eye(E, dtype=jnp.float32), R, axis=1)
    bg2 = bg.reshape(1, E)

    grid = (T // _BT,)
    return pl.pallas_call(
        _moe_body,
        grid=grid,
        in_specs=[
            pl.BlockSpec((_BT, D), lambda i: (i, 0)),
            pl.BlockSpec((D, E * R + E), lambda i: (0, 0)),
            pl.BlockSpec((1, E), lambda i: (0, 0)),
            pl.BlockSpec((E * R, D), lambda i: (0, 0)),
            pl.BlockSpec((E, E * R), lambda i: (0, 0)),
        ],
        out_specs=pl.BlockSpec((_BT, D), lambda i: (i, 0)),
        out_shape=jax.ShapeDtypeStruct((T, D), jnp.float32),
    )(x, m, bg2, b_flat, rep)


# speedup vs baseline: 1.4803x; 1.4803x over previous
import jax
import jax.numpy as jnp
from jax.experimental import pallas as pl

_E = 8
_K = 2
_R = 8
_ALPHA = 32.0
_SCALING = _ALPHA / _R

_BT = 2048


def _moe_body(x_ref, wgt_ref, bg_ref, af_ref, bf_ref, rep_ref, o_ref):
    xv = x_ref[...]
    logits = jnp.dot(xv, wgt_ref[...], preferred_element_type=jnp.float32)
    logits = logits + bg_ref[...]

    col = jax.lax.broadcasted_iota(jnp.int32, logits.shape, 1).astype(jnp.float32)
    neg_inf = jnp.float32(-jnp.inf)
    big = jnp.float32(_E)

    m1 = jnp.max(logits, axis=-1, keepdims=True)
    i1 = jnp.min(jnp.where(logits == m1, col, big), axis=-1, keepdims=True)
    sel1 = col == i1
    l2 = jnp.where(sel1, neg_inf, logits)
    m2 = jnp.max(l2, axis=-1, keepdims=True)
    i2 = jnp.min(jnp.where(l2 == m2, col, big), axis=-1, keepdims=True)
    sel2 = col == i2

    p2 = jnp.exp(m2 - m1)
    w1 = 1.0 / (1.0 + p2)
    w2 = 1.0 - w1
    zero = jnp.float32(0.0)
    w_full = jnp.where(sel1, w1, zero) + jnp.where(sel2, w2, zero)

    h = jnp.dot(xv, af_ref[...], preferred_element_type=jnp.float32)
    w_rep = jnp.dot(w_full, rep_ref[...], preferred_element_type=jnp.float32)
    o_ref[...] = jnp.dot(h * w_rep, bf_ref[...], preferred_element_type=jnp.float32)


@jax.jit
def kernel(x, Wg, bg, A, B):
    T, D = x.shape
    E, R, _ = A.shape
    a_flat = A.reshape(E * R, D).T
    wgt = Wg.T
    b_flat = (B.transpose(0, 2, 1) * jnp.float32(_SCALING)).reshape(E * R, D)
    rep = jnp.repeat(jnp.eye(E, dtype=jnp.float32), R, axis=1)
    bg2 = bg.reshape(1, E)

    grid = (T // _BT,)
    return pl.pallas_call(
        _moe_body,
        grid=grid,
        in_specs=[
            pl.BlockSpec((_BT, D), lambda i: (i, 0)),
            pl.BlockSpec((D, E), lambda i: (0, 0)),
            pl.BlockSpec((1, E), lambda i: (0, 0)),
            pl.BlockSpec((D, E * R), lambda i: (0, 0)),
            pl.BlockSpec((E * R, D), lambda i: (0, 0)),
            pl.BlockSpec((E, E * R), lambda i: (0, 0)),
        ],
        out_specs=pl.BlockSpec((_BT, D), lambda i: (i, 0)),
        out_shape=jax.ShapeDtypeStruct((T, D), jnp.float32),
    )(x, wgt, bg2, a_flat, b_flat, rep)


# speedup vs baseline: 1.4849x; 1.0031x over previous
import jax
import jax.numpy as jnp
from jax.experimental import pallas as pl

_E = 8
_K = 2
_R = 8
_ALPHA = 32.0
_SCALING = _ALPHA / _R

_BT = 2048


def _moe_body(x_ref, wgt_ref, bg_ref, af_ref, bf_ref, rep_ref, o_ref):
    xv = x_ref[...]
    logits = jnp.dot(xv, wgt_ref[...], preferred_element_type=jnp.float32)
    logits = logits + bg_ref[...]

    col = jax.lax.broadcasted_iota(jnp.int32, logits.shape, 1).astype(jnp.float32)
    neg_inf = jnp.float32(-jnp.inf)
    big = jnp.float32(_E)

    m1 = jnp.max(logits, axis=-1, keepdims=True)
    i1 = jnp.min(jnp.where(logits == m1, col, big), axis=-1, keepdims=True)
    sel1 = col == i1
    l2 = jnp.where(sel1, neg_inf, logits)
    m2 = jnp.max(l2, axis=-1, keepdims=True)
    i2 = jnp.min(jnp.where(l2 == m2, col, big), axis=-1, keepdims=True)
    sel2 = col == i2

    p2 = jnp.exp(m2 - m1)
    w1 = 1.0 / (1.0 + p2)
    w2 = 1.0 - w1
    zero = jnp.float32(0.0)
    w_full = jnp.where(sel1, w1, zero) + jnp.where(sel2, w2, zero)

    h = jnp.dot(
        xv.astype(jnp.bfloat16), af_ref[...], preferred_element_type=jnp.float32
    )
    w_rep = jnp.dot(w_full, rep_ref[...], preferred_element_type=jnp.float32)
    g = (h * w_rep).astype(jnp.bfloat16)
    o_ref[...] = jnp.dot(g, bf_ref[...], preferred_element_type=jnp.float32)


@jax.jit
def kernel(x, Wg, bg, A, B):
    T, D = x.shape
    E, R, _ = A.shape
    a_flat = A.reshape(E * R, D).T.astype(jnp.bfloat16)
    wgt = Wg.T
    b_flat = (
        (B.transpose(0, 2, 1) * jnp.float32(_SCALING))
        .reshape(E * R, D)
        .astype(jnp.bfloat16)
    )
    rep = jnp.repeat(jnp.eye(E, dtype=jnp.float32), R, axis=1)
    bg2 = bg.reshape(1, E)

    grid = (T // _BT,)
    return pl.pallas_call(
        _moe_body,
        grid=grid,
        in_specs=[
            pl.BlockSpec((_BT, D), lambda i: (i, 0)),
            pl.BlockSpec((D, E), lambda i: (0, 0)),
            pl.BlockSpec((1, E), lambda i: (0, 0)),
            pl.BlockSpec((D, E * R), lambda i: (0, 0)),
            pl.BlockSpec((E * R, D), lambda i: (0, 0)),
            pl.BlockSpec((E, E * R), lambda i: (0, 0)),
        ],
        out_specs=pl.BlockSpec((_BT, D), lambda i: (i, 0)),
        out_shape=jax.ShapeDtypeStruct((T, D), jnp.float32),
    )(x, wgt, bg2, a_flat, b_flat, rep)
